# baseline (speedup 1.0000x reference)
import jax
import jax.numpy as jnp
from jax import lax
from jax.experimental import pallas as pl
from jax.experimental.pallas import tpu as pltpu
from jax.experimental.pallas import tpu_sc as plsc

MAXLEN = 200
LPAD = 256
EMBED = 64
WIDE = 128
VOCAB = 1_000_000
BATCH = 1024
NW = 32
ROWS_PER_W = BATCH // NW
CH0 = 128
CH1 = MAXLEN - CH0
NCHROW = 2
NCH = ROWS_PER_W * NCHROW
NG = 2
NH = 2
NS = 3
LANES = 16
CPR = EMBED // LANES


def _body(x_hbm, tok_hbm, pos_hbm, out_hbm,
          xidx_v, pos_v, spmem_x, gbufs, hbufs, sidxs, gsems, osems, isems):
    sid = lax.axis_index("s")
    wid = sid * 2 + lax.axis_index("c")
    base = wid * ROWS_PER_W

    pltpu.sync_copy(x_hbm.at[pl.ds(base, ROWS_PER_W)], xidx_v)
    pltpu.sync_copy(xidx_v, spmem_x.at[sid])
    pltpu.sync_copy(pos_hbm, pos_v)

    def halve(t, carry):
        def chunk(c, carry2):
            s = pl.ds(c * LANES, LANES)
            xidx_v[t, s] = xidx_v[t, s] >> 1
            return carry2
        lax.fori_loop(0, LPAD // LANES, chunk, 0, unroll=2)
        return carry
    lax.fori_loop(0, ROWS_PER_W, halve, 0)

    def stage_idx(t):
        return pltpu.async_copy(spmem_x.at[sid, t], sidxs[t % NS],
                                isems[t % NS])

    def fire(c):
        t, h = c // NCHROW, c % NCHROW
        n = CH0 if h == 0 else CH1
        gb = gbufs[c % NG]
        dst = gb if h == 0 else gb.at[pl.ds(0, CH1)]
        return pltpu.async_copy(
            tok_hbm.at[xidx_v.at[t, pl.ds(h * CH0, n)]],
            dst, gsems[c % NG])

    def select_add(c):
        t, h = c // NCHROW, c % NCHROW
        gbuf = gbufs[c % NG]
        hbuf = hbufs[t % NH]
        sidx = sidxs[t % NS]

        n = CH0 if h == 0 else CH1

        def row(r, carry):
            l = h * CH0 + r
            off = (sidx[l] & 1) * EMBED
            for cc in range(CPR):
                src = pl.ds(off + cc * LANES, LANES)
                dst = pl.ds(cc * LANES, LANES)
                hbuf[l, dst] = gbuf[r, src] + pos_v[l, dst]
            return carry
        lax.fori_loop(0, n, row, 0, unroll=2)

    gh = [None] * NG
    oh = [None] * NH
    ih = [None] * NS
    for t in range(min(NS, ROWS_PER_W)):
        ih[t % NS] = stage_idx(t)
    gh[0] = fire(0)
    for c in range(NCH):
        p = c % NG
        t, h = c // NCHROW, c % NCHROW
        if c + 1 < NCH:
            gh[(c + 1) % NG] = fire(c + 1)
        gh[p].wait()
        if h == 0:
            ih[t % NS].wait()
            if oh[t % NH] is not None:
                oh[t % NH].wait()
                oh[t % NH] = None
        select_add(c)
        if h == NCHROW - 1:
            oh[t % NH] = pltpu.async_copy(hbufs[t % NH],
                                          out_hbm.at[base + t],
                                          osems[t % NH])
            if t + NS < ROWS_PER_W:
                ih[t % NS] = stage_idx(t + NS)
    for hh in oh:
        if hh is not None:
            hh.wait()


@jax.jit
def _emb(x, token_wide, pos_table):
    mesh = plsc.VectorSubcoreMesh(core_axis_name="c", subcore_axis_name="s")

    def body(x_hbm, tok_hbm, pos_hbm, out_hbm,
             xidx_v, pos_v, spmem_x,
             g0, g1, h0, h1, s0, s1, s2,
             gs0, gs1, os0, os1, is0, is1, is2):
        _body(x_hbm, tok_hbm, pos_hbm, out_hbm, xidx_v, pos_v, spmem_x,
              (g0, g1), (h0, h1), (s0, s1, s2),
              (gs0, gs1), (os0, os1), (is0, is1, is2))

    run = pl.kernel(
        body,
        out_type=jax.ShapeDtypeStruct((BATCH, MAXLEN, EMBED), jnp.float32),
        mesh=mesh,
        compiler_params=pltpu.CompilerParams(use_tc_tiling_on_sc=True),
        scratch_types=[
            pltpu.VMEM((ROWS_PER_W, LPAD), jnp.int32),
            pltpu.VMEM((MAXLEN, EMBED), jnp.float32),
            pltpu.VMEM_SHARED((16, ROWS_PER_W, LPAD), jnp.int32),
            pltpu.VMEM((CH0, WIDE), jnp.float32),
            pltpu.VMEM((CH0, WIDE), jnp.float32),
            pltpu.VMEM((MAXLEN, EMBED), jnp.float32),
            pltpu.VMEM((MAXLEN, EMBED), jnp.float32),
            pltpu.SMEM((LPAD,), jnp.int32),
            pltpu.SMEM((LPAD,), jnp.int32),
            pltpu.SMEM((LPAD,), jnp.int32),
            pltpu.SemaphoreType.DMA,
            pltpu.SemaphoreType.DMA,
            pltpu.SemaphoreType.DMA,
            pltpu.SemaphoreType.DMA,
            pltpu.SemaphoreType.DMA,
            pltpu.SemaphoreType.DMA,
            pltpu.SemaphoreType.DMA,
        ],
    )
    return run(x, token_wide, pos_table)


def kernel(x, token_table, pos_table):
    xp = jnp.pad(x.astype(jnp.int32), ((0, 0), (0, LPAD - MAXLEN)))
    tok_wide = token_table.reshape(VOCAB * EMBED // WIDE, WIDE)
    return _emb(xp, tok_wide, pos_table)


# speedup vs baseline: 1.4152x; 1.4152x over previous
import jax
import jax.numpy as jnp
from jax import lax
from jax.experimental import pallas as pl
from jax.experimental.pallas import tpu as pltpu
from jax.experimental.pallas import tpu_sc as plsc

MAXLEN = 200
LPAD = 256
EMBED = 64
BATCH = 1024
NW = 32
ROWS_PER_W = BATCH // NW
NBUF = 3
LANES = 16
CPR = EMBED // LANES
SPLIT = 96


def _body(x_hbm, tok_hbm, pos_hbm, out_hbm,
          xidx_v, pos_v, spmem_x, gbufs, sidxs, gsems, gsems2, osems, isems):
    sid = lax.axis_index("s")
    wid = sid * 2 + lax.axis_index("c")
    base = wid * ROWS_PER_W

    pltpu.sync_copy(x_hbm.at[pl.ds(base, ROWS_PER_W)], xidx_v)
    pltpu.sync_copy(xidx_v, spmem_x.at[sid])
    pltpu.sync_copy(pos_hbm, pos_v)

    def stage_idx(t):
        p = t % NBUF
        return pltpu.async_copy(spmem_x.at[sid, t], sidxs[p], isems[p])

    def fire_gathers(t):
        p = t % NBUF
        sidx = sidxs[p]
        gbuf = gbufs[p]
        sem_a = gsems[p]
        sem_b = gsems2[p]

        def one_a(i, carry):
            pltpu.async_copy(tok_hbm.at[sidx[i]], gbuf.at[i], sem_a)
            return carry

        def one_b(i, carry):
            pltpu.async_copy(tok_hbm.at[sidx[i]], gbuf.at[i], sem_b)
            return carry
        lax.fori_loop(0, SPLIT, one_a, 0, unroll=8)
        lax.fori_loop(SPLIT, MAXLEN, one_b, 0, unroll=8)

    def drain_gathers(t):
        p = t % NBUF
        pltpu.make_async_copy(tok_hbm.at[pl.ds(0, SPLIT)],
                              gbufs[p].at[pl.ds(0, SPLIT)],
                              gsems[p]).wait()
        pltpu.make_async_copy(tok_hbm.at[pl.ds(0, MAXLEN - SPLIT)],
                              gbufs[p].at[pl.ds(SPLIT, MAXLEN - SPLIT)],
                              gsems2[p]).wait()

    def add_pos(t):
        p = t % NBUF
        gbuf = gbufs[p]

        def row(r, carry):
            for c in range(CPR):
                s = pl.ds(c * LANES, LANES)
                gbuf[r, s] = gbuf[r, s] + pos_v[r, s]
            return carry
        lax.fori_loop(0, MAXLEN, row, 0, unroll=4)

    ih = [None] * NBUF
    oh = [None] * NBUF
    for t in range(min(NBUF, ROWS_PER_W)):
        ih[t % NBUF] = stage_idx(t)
    ih[0].wait()
    fire_gathers(0)

    for t in range(ROWS_PER_W):
        p = t % NBUF
        q = (t + 1) % NBUF
        if t + 1 < ROWS_PER_W:
            if oh[q] is not None:
                oh[q].wait()
                oh[q] = None
            ih[q].wait()
            fire_gathers(t + 1)
        if t + NBUF < ROWS_PER_W:
            ih[p] = stage_idx(t + NBUF)
        drain_gathers(t)
        add_pos(t)
        oh[p] = pltpu.async_copy(gbufs[p], out_hbm.at[base + t], osems[p])
    for h in oh:
        if h is not None:
            h.wait()


@jax.jit
def _emb(x, token_table, pos_table):
    mesh = plsc.VectorSubcoreMesh(core_axis_name="c", subcore_axis_name="s")

    def body(x_hbm, tok_hbm, pos_hbm, out_hbm,
             xidx_v, pos_v, spmem_x,
             g0, g1, g2, s0, s1, s2,
             gs0, gs1, gs2, gt0, gt1, gt2,
             os0, os1, os2, is0, is1, is2):
        _body(x_hbm, tok_hbm, pos_hbm, out_hbm, xidx_v, pos_v, spmem_x,
              (g0, g1, g2), (s0, s1, s2),
              (gs0, gs1, gs2), (gt0, gt1, gt2), (os0, os1, os2),
              (is0, is1, is2))

    run = pl.kernel(
        body,
        out_type=jax.ShapeDtypeStruct((BATCH, MAXLEN, EMBED), jnp.float32),
        mesh=mesh,
        compiler_params=pltpu.CompilerParams(use_tc_tiling_on_sc=True),
        scratch_types=[
            pltpu.VMEM((ROWS_PER_W, LPAD), jnp.int32),
            pltpu.VMEM((MAXLEN, EMBED), jnp.float32),
            pltpu.VMEM_SHARED((16, ROWS_PER_W, LPAD), jnp.int32),
            pltpu.VMEM((MAXLEN, EMBED), jnp.float32),
            pltpu.VMEM((MAXLEN, EMBED), jnp.float32),
            pltpu.VMEM((MAXLEN, EMBED), jnp.float32),
            pltpu.SMEM((LPAD,), jnp.int32),
            pltpu.SMEM((LPAD,), jnp.int32),
            pltpu.SMEM((LPAD,), jnp.int32),
            pltpu.SemaphoreType.DMA,
            pltpu.SemaphoreType.DMA,
            pltpu.SemaphoreType.DMA,
            pltpu.SemaphoreType.DMA,
            pltpu.SemaphoreType.DMA,
            pltpu.SemaphoreType.DMA,
            pltpu.SemaphoreType.DMA,
            pltpu.SemaphoreType.DMA,
            pltpu.SemaphoreType.DMA,
            pltpu.SemaphoreType.DMA,
            pltpu.SemaphoreType.DMA,
            pltpu.SemaphoreType.DMA,
        ],
    )
    return run(x, token_table, pos_table)


def kernel(x, token_table, pos_table):
    xp = jnp.pad(x.astype(jnp.int32), ((0, 0), (0, LPAD - MAXLEN)))
    return _emb(xp, token_table, pos_table)
